# baseline (speedup 1.0000x reference)
import functools

import jax
import jax.numpy as jnp
from jax import lax
from jax.experimental import pallas as pl
from jax.experimental.pallas import tpu as pltpu
from jax.experimental.pallas import tpu_sc as plsc

_VOCAB = 1000000
_VOCAB_PAD = 1007616
_EMB = 16
_B = 16384
_L = 200

_NC = 2
_NS = 16
_NW = _NC * _NS
_SPW = _B // _NW
_GRP = 16
_NGRP = _SPW // _GRP
_CHUNK = _GRP * _L


def _scores_body(t_ref, w_ref, b_ref, o_ref):
    m = t_ref[:] * w_ref[:]
    o_ref[:] = jnp.sum(m, axis=1) + b_ref[0]


def _vocab_scores(table, W, b):
    blk = 8192
    return pl.pallas_call(
        _scores_body,
        grid=(_VOCAB_PAD // blk,),
        in_specs=[
            pl.BlockSpec((blk, _EMB), lambda i: (i, 0)),
            pl.BlockSpec((1, _EMB), lambda i: (0, 0)),
            pl.BlockSpec(memory_space=pltpu.SMEM),
        ],
        out_specs=pl.BlockSpec((blk,), lambda i: (i,)),
        out_shape=jax.ShapeDtypeStruct((_VOCAB_PAD,), jnp.float32),
    )(table, W, b)


def _pool_body(xf_hbm, scores_hbm, out_hbm, idx_v, vals_v, res_v, sem):
    wid = lax.axis_index("s") * _NC + lax.axis_index("c")
    base_sample = wid * _SPW
    lane = lax.iota(jnp.int32, 16)
    gidx0 = lane * _L

    def group_body(g, carry):
        s0 = base_sample + g * _GRP
        pltpu.sync_copy(xf_hbm.at[pl.ds(s0 * _L, _CHUNK)], idx_v)
        pltpu.async_copy(scores_hbm.at[idx_v], vals_v, sem).wait()
        accs = [jnp.zeros((16,), jnp.float32) for _ in range(4)]
        for j in range(_L):
            v = plsc.load_gather(vals_v, [gidx0 + j])
            accs[j % 4] = accs[j % 4] + v
        tot = (accs[0] + accs[1]) + (accs[2] + accs[3])
        z = tot * (1.0 / _L)
        res_v[...] = 1.0 / (1.0 + jnp.exp(-z))
        pltpu.sync_copy(res_v, out_hbm.at[pl.ds(s0, _GRP)])
        return carry

    lax.fori_loop(0, _NGRP, group_body, 0)


def _pool(x_flat, scores):
    mesh = plsc.VectorSubcoreMesh(core_axis_name="c", subcore_axis_name="s")
    return pl.kernel(
        _pool_body,
        out_type=jax.ShapeDtypeStruct((_B,), jnp.float32),
        mesh=mesh,
        compiler_params=pltpu.CompilerParams(needs_layout_passes=False),
        scratch_types=[
            pltpu.VMEM((_CHUNK,), jnp.int32),
            pltpu.VMEM((_CHUNK,), jnp.float32),
            pltpu.VMEM((_GRP,), jnp.float32),
            pltpu.SemaphoreType.DMA,
        ],
    )(x_flat, scores)


def kernel(x, table, W, b):
    scores = _vocab_scores(table, W, b)
    p = _pool(x.reshape(_B * _L), scores)
    return jnp.round(p.reshape(_B, 1), decimals=4)


# speedup vs baseline: 1.4130x; 1.4130x over previous
import functools

import jax
import jax.numpy as jnp
from jax import lax
from jax.experimental import pallas as pl
from jax.experimental.pallas import tpu as pltpu
from jax.experimental.pallas import tpu_sc as plsc

_VOCAB = 1000000
_VOCAB_PAD = 1007616
_EMB = 16
_B = 16384
_L = 200

_NC = 2
_NS = 16
_NW = _NC * _NS
_SPW = _B // _NW
_GRP = 16
_NGRP = _SPW // _GRP
_CHUNK = _GRP * _L


def _scores_body(t_ref, w_ref, b_ref, o_ref):
    d = lax.dot_general(
        w_ref[:], t_ref[:],
        dimension_numbers=(((1,), (1,)), ((), ())),
        preferred_element_type=jnp.float32,
    )
    o_ref[:] = d.reshape(o_ref.shape) + b_ref[0]


def _vocab_scores(table, W, b):
    blk = 8192
    return pl.pallas_call(
        _scores_body,
        grid=(_VOCAB_PAD // blk,),
        in_specs=[
            pl.BlockSpec((blk, _EMB), lambda i: (i, 0)),
            pl.BlockSpec((1, _EMB), lambda i: (0, 0)),
            pl.BlockSpec(memory_space=pltpu.SMEM),
        ],
        out_specs=pl.BlockSpec((blk,), lambda i: (i,)),
        out_shape=jax.ShapeDtypeStruct((_VOCAB_PAD,), jnp.float32),
    )(table, W, b)


def _pool_body(xf_hbm, scores_hbm, out_hbm, idx_v, vals_v, res_v, sem):
    wid = lax.axis_index("s") * _NC + lax.axis_index("c")
    base_sample = wid * _SPW
    lane = lax.iota(jnp.int32, 16)
    gidx0 = lane * _L

    def group_body(g, carry):
        s0 = base_sample + g * _GRP
        pltpu.sync_copy(xf_hbm.at[pl.ds(s0 * _L, _CHUNK)], idx_v)
        pltpu.async_copy(scores_hbm.at[idx_v], vals_v, sem).wait()
        accs = [jnp.zeros((16,), jnp.float32) for _ in range(4)]
        for j in range(_L):
            v = plsc.load_gather(vals_v, [gidx0 + j])
            accs[j % 4] = accs[j % 4] + v
        tot = (accs[0] + accs[1]) + (accs[2] + accs[3])
        z = tot * (1.0 / _L)
        res_v[...] = 1.0 / (1.0 + jnp.exp(-z))
        pltpu.sync_copy(res_v, out_hbm.at[pl.ds(s0, _GRP)])
        return carry

    lax.fori_loop(0, _NGRP, group_body, 0)


def _pool(x_flat, scores):
    mesh = plsc.VectorSubcoreMesh(core_axis_name="c", subcore_axis_name="s")
    return pl.kernel(
        _pool_body,
        out_type=jax.ShapeDtypeStruct((_B,), jnp.float32),
        mesh=mesh,
        compiler_params=pltpu.CompilerParams(needs_layout_passes=False),
        scratch_types=[
            pltpu.VMEM((_CHUNK,), jnp.int32),
            pltpu.VMEM((_CHUNK,), jnp.float32),
            pltpu.VMEM((_GRP,), jnp.float32),
            pltpu.SemaphoreType.DMA,
        ],
    )(x_flat, scores)


def kernel(x, table, W, b):
    scores = _vocab_scores(table, W, b)
    p = _pool(x.reshape(_B * _L), scores)
    return jnp.round(p.reshape(_B, 1), decimals=4)
